# baseline (speedup 1.0000x reference)
import jax
import jax.numpy as jnp
from jax.experimental import pallas as pl
from jax.experimental.pallas import tpu as pltpu

_NUM_CELLS = 8192
_CELL_DIM = 32
_BS = 512
_LANE = 128


def _combine(m, ti):
    lane = jax.lax.broadcasted_iota(
        jnp.int32, (_BS, _LANE), 1).astype(jnp.float32)
    full = ti * jnp.float32(_LANE) + lane
    rm = jnp.max(m, axis=1, keepdims=True)
    idx = jnp.min(jnp.where(m == rm, full, jnp.float32(_NUM_CELLS)), axis=1)
    return idx.astype(jnp.int32)


def _pc_argmax_kernel(x_ref, pc_ref, out_ref, last_ref, m_s, ti_s):
    i = pl.program_id(0)
    nsteps = pl.num_programs(0)

    sl_prev = jax.lax.rem(i + 1, 2)
    out_ref[...] = _combine(m_s[sl_prev], ti_s[sl_prev])

    xb = x_ref[...]
    nt = _NUM_CELLS // _LANE
    m = None
    ti = jnp.zeros((_BS, _LANE), jnp.float32)
    for j in range(nt):
        pcj = pc_ref[j * _LANE:(j + 1) * _LANE, :]
        sj = jax.lax.dot_general(
            xb, pcj,
            dimension_numbers=(((1,), (1,)), ((), ())),
            preferred_element_type=jnp.float32,
        )
        if j == 0:
            m = sj
        else:
            g = sj > m
            m = jnp.maximum(m, sj)
            ti = jnp.where(g, jnp.float32(j), ti)
    sl = jax.lax.rem(i, 2)
    m_s[sl] = m
    ti_s[sl] = ti

    @pl.when(i == nsteps - 1)
    def _combine_last():
        last_ref[...] = _combine(m, ti)


def kernel(x, placeCells):
    states = jnp.reshape(x, (-1, _CELL_DIM))
    n = states.shape[0]
    nsteps = n // _BS
    head, last = pl.pallas_call(
        _pc_argmax_kernel,
        grid=(nsteps,),
        in_specs=[
            pl.BlockSpec((_BS, _CELL_DIM), lambda i: (i, 0)),
            pl.BlockSpec((_NUM_CELLS, _CELL_DIM), lambda i: (0, 0)),
        ],
        out_specs=[
            pl.BlockSpec((_BS,), lambda i: (jnp.maximum(i - 1, 0),)),
            pl.BlockSpec((_BS,), lambda i: (0,)),
        ],
        out_shape=[
            jax.ShapeDtypeStruct((n - _BS,), jnp.int32),
            jax.ShapeDtypeStruct((_BS,), jnp.int32),
        ],
        scratch_shapes=[
            pltpu.VMEM((2, _BS, _LANE), jnp.float32),
            pltpu.VMEM((2, _BS, _LANE), jnp.float32),
        ],
    )(states, placeCells)
    return jnp.concatenate([head, last])


# speedup vs baseline: 1.0183x; 1.0183x over previous
import jax
import jax.numpy as jnp
from jax.experimental import pallas as pl
from jax.experimental.pallas import tpu as pltpu

_NUM_CELLS = 8192
_CELL_DIM = 32
_BS = 512
_LANE = 128


def _pc_argmax_kernel(x_ref, pc_ref, out_ref):
    xb = x_ref[...]
    nt = _NUM_CELLS // _LANE
    m = None
    ti = jnp.zeros((_BS, _LANE), jnp.float32)
    for j in range(nt):
        pcj = pc_ref[j * _LANE:(j + 1) * _LANE, :]
        sj = jax.lax.dot_general(
            xb, pcj,
            dimension_numbers=(((1,), (1,)), ((), ())),
            preferred_element_type=jnp.float32,
        )
        if j == 0:
            m = sj
        else:
            g = sj > m
            m = jnp.maximum(m, sj)
            ti = jnp.where(g, jnp.float32(j), ti)
    lane = jax.lax.broadcasted_iota(
        jnp.int32, (_BS, _LANE), 1).astype(jnp.float32)
    full = ti * jnp.float32(_LANE) + lane
    rm = jnp.max(m, axis=1, keepdims=True)
    idx = jnp.min(jnp.where(m == rm, full, jnp.float32(_NUM_CELLS)), axis=1)
    out_ref[...] = idx.astype(jnp.int32)


def kernel(x, placeCells):
    states = jnp.reshape(x, (-1, _CELL_DIM))
    n = states.shape[0]
    return pl.pallas_call(
        _pc_argmax_kernel,
        grid=(n // _BS,),
        in_specs=[
            pl.BlockSpec((_BS, _CELL_DIM), lambda i: (i, 0)),
            pl.BlockSpec((_NUM_CELLS, _CELL_DIM), lambda i: (0, 0)),
        ],
        out_specs=pl.BlockSpec((_BS,), lambda i: (i,)),
        out_shape=jax.ShapeDtypeStruct((n,), jnp.int32),
        compiler_params=pltpu.CompilerParams(
            dimension_semantics=("parallel",)),
    )(states, placeCells)
